# Initial kernel scaffold; baseline (speedup 1.0000x reference)
import jax
import jax.numpy as jnp
from jax.experimental import pallas as pl


def kernel(pred_boxes, pred_confidence, pred_keypoints, target_boxes, target_keypoints):
    raise NotImplementedError("write your pallas kernel here")


# speedup vs baseline: 1.9541x; 1.9541x over previous
import functools

import jax
import jax.numpy as jnp
from jax import lax
from jax.experimental import pallas as pl
from jax.experimental.pallas import tpu as pltpu
from jax.experimental.pallas import tpu_sc as plsc

GRID = 128
NOBJ = 100
NOBJ_P = 128
ANCHOR_W = 4.0
ANCHOR_H = 4.0
EMPTY_WEIGHT = 5.0


def _prep_body(tb_ref, idx_ref, tv_ref):
    tb = tb_ref[...]
    g = jnp.float32(GRID)
    t_x = tb[..., 0] * g
    t_y = tb[..., 1] * g
    gif = jnp.floor(t_x)
    gjf = jnp.floor(t_y)
    gii = gif.astype(jnp.int32)
    gjj = gjf.astype(jnp.int32)
    bsz = tb.shape[0]
    o = lax.broadcasted_iota(jnp.int32, (bsz, NOBJ_P), 1)
    b = lax.broadcasted_iota(jnp.int32, (bsz, NOBJ_P), 0)
    validm = o < NOBJ
    cell = jnp.where(validm, gjj * GRID + gii, -1 - o)
    eq = cell[:, :, None] == cell[:, None, :]
    later = (lax.broadcasted_iota(jnp.int32, (bsz, NOBJ_P, NOBJ_P), 2)
             > lax.broadcasted_iota(jnp.int32, (bsz, NOBJ_P, NOBJ_P), 1))
    dup = jnp.any(eq & later, axis=2)
    winner = jnp.where(validm & ~dup, 1.0, 0.0).astype(jnp.float32)

    idx_ref[...] = jnp.where(validm, b * (GRID * GRID) + cell, 0)

    tx = t_x - gif
    ty = t_y - gjf
    tz = tb[..., 2]
    tw = jnp.log(tb[..., 3] * g / ANCHOR_W + 1e-16)
    th = jnp.log(tb[..., 4] * g / ANCHOR_H + 1e-16)
    td = jnp.log(tb[..., 5] + 1e-16)
    zero = jnp.zeros_like(tx)
    tv_ref[...] = jnp.stack(
        [tx, ty, tz, tw, th, td,
         t_x, t_y, tb[..., 3] * g, tb[..., 4] * g,
         gif, gjf, winner, zero, zero, zero], axis=-1)


def _run_prep(tb_pad):
    bsz = tb_pad.shape[0]
    return pl.pallas_call(
        _prep_body,
        out_shape=(
            jax.ShapeDtypeStruct((bsz, NOBJ_P), jnp.int32),
            jax.ShapeDtypeStruct((bsz, NOBJ_P, 16), jnp.float32),
        ),
    )(tb_pad)


def _sc_gather_body(nw, b_per_w,
                    idx_hbm, kp_hbm, box_hbm, conf_hbm,
                    okp, obox, oconf,
                    idx_v, kp_v, box_v, conf_v, sem):
    wid = lax.axis_index("s") * 2 + lax.axis_index("c")
    base = wid * b_per_w
    pltpu.sync_copy(idx_hbm.at[pl.ds(base, b_per_w)], idx_v)
    pltpu.async_copy(kp_hbm.at[idx_v], kp_v, sem).wait()
    pltpu.async_copy(box_hbm.at[idx_v], box_v, sem).wait()
    pltpu.async_copy(conf_hbm.at[idx_v], conf_v, sem).wait()
    pltpu.sync_copy(kp_v, okp.at[pl.ds(base, b_per_w)])
    pltpu.sync_copy(box_v, obox.at[pl.ds(base, b_per_w)])
    pltpu.sync_copy(conf_v, oconf.at[pl.ds(base, b_per_w)])


def _run_gather(flat_idx, kp_tab, box_tab, conf_tab):
    return (jnp.take(kp_tab, flat_idx, axis=0),
            jnp.take(box_tab, flat_idx, axis=0),
            jnp.take(conf_tab, flat_idx, axis=0))


def _run_gather_sc(flat_idx, kp_tab, box_tab, conf_tab):
    n = flat_idx.shape[0]
    nkey = kp_tab.shape[1]
    info = plsc.get_sparse_core_info()
    nw = info.num_cores * info.num_subcores
    b_per_w = n // nw
    mesh = plsc.VectorSubcoreMesh(core_axis_name="c", subcore_axis_name="s")
    kern = functools.partial(
        pl.kernel,
        mesh=mesh,
        out_type=(
            jax.ShapeDtypeStruct((n, nkey), jnp.float32),
            jax.ShapeDtypeStruct((n, 6), jnp.float32),
            jax.ShapeDtypeStruct((n, 1), jnp.float32),
        ),
        scratch_types=[
            pltpu.VMEM((b_per_w,), jnp.int32),
            pltpu.VMEM((b_per_w, nkey), jnp.float32),
            pltpu.VMEM((b_per_w, 6), jnp.float32),
            pltpu.VMEM((b_per_w, 1), jnp.float32),
            pltpu.SemaphoreType.DMA,
        ],
    )(functools.partial(_sc_gather_body, nw, b_per_w))
    return kern(flat_idx, kp_tab, box_tab, conf_tab)


def _loss_body(nsteps, ntot,
               conf_ref, gbox_ref, gconf_ref, gkp_ref, tkp_ref,
               tv_blk_ref, tv_full_ref, out_ref, acc_ref):
    i = pl.program_id(0)

    @pl.when(i == 0)
    def _init():
        for k in range(6):
            acc_ref[k] = 0.0

    sp = jax.nn.softplus
    tvb = tv_blk_ref[...]
    w = tvb[:, 12]
    gbox = gbox_ref[...]
    gconf = gconf_ref[...][:, 0]

    d_box = (jnp.abs(gbox[:, 0] - tvb[:, 0]) + jnp.abs(gbox[:, 1] - tvb[:, 1])
             + jnp.abs(gbox[:, 3] - tvb[:, 3]) + jnp.abs(gbox[:, 4] - tvb[:, 4])
             + jnp.abs(gbox[:, 2] - tvb[:, 2]) + jnp.abs(gbox[:, 5] - tvb[:, 5]))
    s_box = jnp.sum(d_box * w)
    s_kp = jnp.sum(jnp.sum(jnp.abs(gkp_ref[...] - tkp_ref[...]), axis=1) * w)
    s_n = jnp.sum(w)
    s_corr = jnp.sum(w * (EMPTY_WEIGHT * sp(-gconf) - sp(gconf)))
    s_base = jnp.sum(sp(conf_ref[...]))

    px = gbox[:, 0] + tvb[:, 10]
    py = gbox[:, 1] + tvb[:, 11]
    pw = jnp.exp(gbox[:, 3]) * ANCHOR_W
    ph = jnp.exp(gbox[:, 4]) * ANCHOR_H
    vP = w > 0
    px1 = jnp.where(vP, px - 0.5 * pw, 0.0)[:, None]
    px2 = jnp.where(vP, px + 0.5 * pw, 1.0)[:, None]
    py1 = jnp.where(vP, py - 0.5 * ph, 0.0)[:, None]
    py2 = jnp.where(vP, py + 0.5 * ph, 1.0)[:, None]
    areap = jnp.where(vP, pw * ph, 1.0)[:, None]
    wP = w[:, None]

    tvf = tv_full_ref[...]
    wT = tvf[:, 12]
    vT = wT > 0
    ttx, tty, ttw, tth = tvf[:, 6], tvf[:, 7], tvf[:, 8], tvf[:, 9]
    tx1 = jnp.where(vT, ttx - 0.5 * ttw, 0.0)[None, :]
    tx2 = jnp.where(vT, ttx + 0.5 * ttw, 1.0)[None, :]
    ty1 = jnp.where(vT, tty - 0.5 * tth, 0.0)[None, :]
    ty2 = jnp.where(vT, tty + 0.5 * tth, 1.0)[None, :]
    areat = jnp.where(vT, ttw * tth, 1.0)[None, :]
    wTr = wT[None, :]

    ix = jnp.maximum(jnp.minimum(px2, tx2) - jnp.maximum(px1, tx1), 0.0)
    iy = jnp.maximum(jnp.minimum(py2, ty2) - jnp.maximum(py1, ty1), 0.0)
    inter = ix * iy
    union = areap + areat - inter
    s_iou = jnp.sum((union - inter) / union * (wP * wTr))

    acc_ref[0] += s_box
    acc_ref[1] += s_kp
    acc_ref[2] += s_corr
    acc_ref[3] += s_base
    acc_ref[4] += s_iou
    acc_ref[5] += s_n

    @pl.when(i == nsteps - 1)
    def _fin():
        n = acc_ref[5]
        out_ref[...] = jnp.stack([
            acc_ref[0] / n,
            acc_ref[1] / n,
            (acc_ref[3] + acc_ref[2]) / jnp.float32(ntot),
            acc_ref[4] / n,
        ])


def _run_losses(conf2d, gbox, gconf, gkp, tkp2d, tv2d):
    nent = tv2d.shape[0]
    nkey = tkp2d.shape[1]
    rows = 128
    nsteps = nent // rows
    ntot = conf2d.shape[0] * conf2d.shape[1]
    blk = lambda i: (i, 0)
    return pl.pallas_call(
        functools.partial(_loss_body, nsteps, ntot),
        grid=(nsteps,),
        in_specs=[
            pl.BlockSpec((rows, conf2d.shape[1]), blk),
            pl.BlockSpec((rows, 6), blk),
            pl.BlockSpec((rows, 1), blk),
            pl.BlockSpec((rows, nkey), blk),
            pl.BlockSpec((rows, nkey), blk),
            pl.BlockSpec((rows, 16), blk),
            pl.BlockSpec((nent, 16), lambda i: (0, 0)),
        ],
        out_specs=pl.BlockSpec((4,), lambda i: (0,)),
        out_shape=jax.ShapeDtypeStruct((4,), jnp.float32),
        scratch_shapes=[pltpu.SMEM((8,), jnp.float32)],
    )(conf2d, gbox, gconf, gkp, tkp2d, tv2d, tv2d)


def kernel(pred_boxes, pred_confidence, pred_keypoints, target_boxes, target_keypoints):
    bsz, g = pred_boxes.shape[0], pred_boxes.shape[1]
    nkey = pred_keypoints.shape[-1]

    tb_pad = jnp.pad(target_boxes, ((0, 0), (0, NOBJ_P - NOBJ), (0, 0)))
    idx2d, tv = _run_prep(tb_pad)

    flat_idx = idx2d.reshape(bsz * NOBJ_P)
    kp_tab = pred_keypoints.reshape(bsz * g * g, nkey)
    box_tab = pred_boxes.reshape(bsz * g * g, 6)
    conf_tab = pred_confidence.reshape(bsz * g * g, 1)
    gkp, gbox, gconf = _run_gather(flat_idx, kp_tab, box_tab, conf_tab)

    tkp_pad = jnp.pad(target_keypoints, ((0, 0), (0, NOBJ_P - NOBJ), (0, 0)))
    tkp2d = tkp_pad.reshape(bsz * NOBJ_P, nkey)
    tv2d = tv.reshape(bsz * NOBJ_P, 16)
    conf2d = pred_confidence.reshape(bsz * g * g // 128, 128)
    return _run_losses(conf2d, gbox, gconf, gkp, tkp2d, tv2d)
